# baseline (speedup 1.0000x reference)
import functools

import jax
import jax.numpy as jnp
from jax import lax
from jax.experimental import pallas as pl
from jax.experimental.pallas import tpu as pltpu
from jax.experimental.pallas import tpu_sc as plsc

BATCH = 16384
DIM = 64
NC = 2
NS = 16
LANES = 16
NW = NC * NS
B_PER_W = BATCH // NW
CHUNK = 128
NCHUNK = B_PER_W // CHUNK


def _sc_body(uid_hbm, iid_hbm, uw_hbm, iw_hbm,
             ratings_hbm, uf_hbm, if_hbm,
             uidx, iidx, urows, irows, tbuf, rat, sem_u, sem_i):
    wid = lax.axis_index("s") * NC + lax.axis_index("c")
    base = wid * B_PER_W

    for j in range(NCHUNK):
        pltpu.sync_copy(uid_hbm.at[pl.ds(base + j * CHUNK, CHUNK)], uidx.at[j])
        pltpu.sync_copy(iid_hbm.at[pl.ds(base + j * CHUNK, CHUNK)], iidx.at[j])

    copies = []
    for j in range(NCHUNK):
        copies.append(pltpu.async_copy(
            uw_hbm.at[uidx.at[j]], urows.at[pl.ds(j * CHUNK, CHUNK)], sem_u))
        copies.append(pltpu.async_copy(
            iw_hbm.at[iidx.at[j]], irows.at[pl.ds(j * CHUNK, CHUNK)], sem_i))
    for c in copies:
        c.wait()

    iota16 = lax.iota(jnp.int32, LANES)

    def group(g, carry):
        base_r = g * LANES
        for b in range(LANES):
            row = base_r + b
            acc = urows[row, pl.ds(0, LANES)] * irows[row, pl.ds(0, LANES)]
            for k in range(1, DIM // LANES):
                acc = acc + (urows[row, pl.ds(k * LANES, LANES)]
                             * irows[row, pl.ds(k * LANES, LANES)])
            plsc.store_scatter(tbuf, [iota16 * LANES + b], acc)
        rv = tbuf[pl.ds(0, LANES)]
        for j in range(1, LANES):
            rv = rv + tbuf[pl.ds(j * LANES, LANES)]
        rat[pl.ds(base_r, LANES)] = rv
        return carry

    lax.fori_loop(0, B_PER_W // LANES, group, 0)

    pltpu.sync_copy(urows, uf_hbm.at[pl.ds(base, B_PER_W)])
    pltpu.sync_copy(irows, if_hbm.at[pl.ds(base, B_PER_W)])
    pltpu.sync_copy(rat, ratings_hbm.at[pl.ds(base, B_PER_W)])


@jax.jit
def kernel(user_ids, item_ids, user_weight, item_weight):
    mesh = plsc.VectorSubcoreMesh(core_axis_name="c", subcore_axis_name="s")
    out_type = (
        jax.ShapeDtypeStruct((BATCH,), jnp.float32),
        jax.ShapeDtypeStruct((BATCH, DIM), jnp.float32),
        jax.ShapeDtypeStruct((BATCH, DIM), jnp.float32),
    )
    scratch = [
        pltpu.VMEM((NCHUNK, CHUNK), jnp.int32),
        pltpu.VMEM((NCHUNK, CHUNK), jnp.int32),
        pltpu.VMEM((B_PER_W, DIM), jnp.float32),
        pltpu.VMEM((B_PER_W, DIM), jnp.float32),
        pltpu.VMEM((LANES * LANES,), jnp.float32),
        pltpu.VMEM((B_PER_W,), jnp.float32),
        pltpu.SemaphoreType.DMA,
        pltpu.SemaphoreType.DMA,
    ]
    run = pl.kernel(_sc_body, out_type=out_type, mesh=mesh,
                    scratch_types=scratch,
                    compiler_params=pltpu.CompilerParams(
                        needs_layout_passes=False,
                        use_tc_tiling_on_sc=False))
    return run(user_ids.astype(jnp.int32), item_ids.astype(jnp.int32),
               user_weight, item_weight)
